# baseline (speedup 1.0000x reference)
import jax
import jax.numpy as jnp
from jax import lax
from jax.experimental import pallas as pl
from jax.experimental.pallas import tpu as pltpu

_LAYER_DIMS = ((3, 16), (16, 32), (32, 16), (16, 3))
_RELUS = (True, True, True, False)
_LANE = 128
_F = 4
_B = 128

_KH = [_F * 16 if cin * _F < 16 else _F * cin for cin, _ in _LAYER_DIMS]
_KH[0] = 16
_SEG = [16, 16, 32, 16]
_KTOT = [_KH[i] + 2 * _SEG[i] + 16 for i in range(4)]


def _align(v, m):
    return -(-v // m) * m


def _build_weights(w_packed, b_packed):
    ws = []
    for li, (cin, cout) in enumerate(_LAYER_DIMS):
        wt = w_packed[3 * li:3 * li + 3, :cout, :cin]
        last = li == len(_LAYER_DIMS) - 1
        om = 3 if last else cout
        m = _align(_F * om, 16)
        kh, seg = _KH[li], _SEG[li]
        w = jnp.zeros((m, _KTOT[li]), jnp.float32)
        for r in range(_F):
            ro = r * om
            for t in range(3):
                rp = r + t - 1
                if 0 <= rp < _F:
                    w = w.at[ro:ro + cout,
                             rp * cin:rp * cin + cin].set(wt[t])
            if r == 0:
                off = kh + ((_F - 1) * cin if li == 0 else 0)
                w = w.at[ro:ro + cout, off:off + cin].set(wt[0])
            if r == _F - 1:
                off = kh + seg + 0
                w = w.at[ro:ro + cout, off:off + cin].set(wt[2])
        bcol = jnp.tile(b_packed[li, :cout, 0], (_F,))
        rows = (om * jnp.arange(_F)[:, None]
                + jnp.arange(cout)[None, :]).reshape(-1)
        w = w.at[rows, kh + 2 * seg].set(bcol)
        ws.append(w.astype(jnp.bfloat16))
    return ws


def _make_body(width):
    bf = jnp.bfloat16
    n_l = len(_LAYER_DIMS)

    def body(x_ref, w1, w2, w3, w4, o_ref, s1, s2, s3, s4):
        wrefs = (w1, w2, w3, w4)
        srefs = (s1, s2, s3, s4)

        @pl.when(pl.program_id(0) == 0)
        def _init():
            for li in range(n_l):
                s = srefs[li]
                kh, seg = _KH[li], _SEG[li]
                s[...] = jnp.zeros(s.shape, bf)
                s[kh + 2 * seg:kh + 2 * seg + 1] = jnp.ones((1, width), bf)

        def put(li, hval):
            s = srefs[li]
            kh, seg = _KH[li], _SEG[li]
            cin = _LAYER_DIMS[li][0]
            s[0:hval.shape[0]] = hval
            if li == 0:
                ep_src, en_src = hval, hval
                ep_rows, en_rows = hval.shape[0], hval.shape[0]
            else:
                ep_src = hval[(_F - 1) * cin:_F * cin]
                en_src = hval[:cin]
                ep_rows, en_rows = cin, cin
            s[kh:kh + ep_rows, _B:] = ep_src[:, :width - _B]
            s[kh + seg:kh + seg + en_rows, :width - _B] = en_src[:, _B:]

        put(0, x_ref[0])
        for li in range(n_l):
            y = jnp.dot(wrefs[li][...], srefs[li][...],
                        preferred_element_type=jnp.float32)
            if _RELUS[li]:
                put(li + 1, jnp.maximum(y.astype(bf), jnp.bfloat16(0)))
            else:
                o_ref[0] = y[:_F * 3].astype(bf)
    return body


_CHUNKS = 2


def _run_chunk(xp, ws, body, c, q, width):
    n_steps = xp.shape[0] // _B
    xt = (xp.astype(jnp.bfloat16)
          .reshape(n_steps, _B, c, q, _F)
          .transpose(0, 4, 2, 3, 1)
          .reshape(n_steps, _F * c, width))

    full = lambda a: pl.BlockSpec(a.shape, lambda i: (0,) * a.ndim)
    out = pl.pallas_call(
        body,
        out_shape=jax.ShapeDtypeStruct((n_steps, _F * c, width), jnp.bfloat16),
        grid=(n_steps,),
        in_specs=[pl.BlockSpec((1, _F * c, width), lambda i: (i, 0, 0))]
        + [full(w) for w in ws],
        out_specs=pl.BlockSpec((1, _F * c, width), lambda i: (i, 0, 0)),
        scratch_shapes=[pltpu.VMEM((k, width), jnp.bfloat16) for k in _KTOT],
        compiler_params=pltpu.CompilerParams(
            dimension_semantics=("arbitrary",),
            vmem_limit_bytes=60 * 1024 * 1024,
        ),
    )(xt, *ws)

    return (out.reshape(n_steps, _F, c, q, _B)
            .transpose(0, 4, 2, 3, 1)
            .reshape(n_steps * _B, c, q * _F))


def kernel(x_ncl, w_packed, b_packed):
    n, c, l = x_ncl.shape
    l_pad = _align(l, _F)
    n_chunk = _B * _CHUNKS
    n_pad = _align(n, n_chunk)

    xp = x_ncl
    if n_pad != n or l_pad != l:
        xp = jnp.pad(x_ncl, ((0, n_pad - n), (0, 0), (0, l_pad - l)))

    q = l_pad // _F
    width = q * _B

    ws = _build_weights(w_packed, b_packed)
    body = _make_body(width)

    csize = n_pad // _CHUNKS
    outs = [_run_chunk(xp[i * csize:(i + 1) * csize], ws, body, c, q, width)
            for i in range(_CHUNKS)]
    out = outs[0] if _CHUNKS == 1 else jnp.concatenate(outs, axis=0)
    out = out.astype(x_ncl.dtype)
    if n_pad != n or l_pad != l:
        out = out[:n, :, :l]
    return out


# speedup vs baseline: 1.0724x; 1.0724x over previous
import jax
import jax.numpy as jnp
from jax import lax
from jax.experimental import pallas as pl
from jax.experimental.pallas import tpu as pltpu

_LAYER_DIMS = ((3, 16), (16, 32), (32, 16), (16, 3))
_RELUS = (True, True, True, False)
_LANE = 128
_F = 4
_B = 128

_KH = [_F * 16 if cin * _F < 16 else _F * cin for cin, _ in _LAYER_DIMS]
_KH[0] = 16
_SEG = [16, 16, 32, 16]
_KTOT = [_KH[i] + 2 * _SEG[i] + 16 for i in range(4)]


def _align(v, m):
    return -(-v // m) * m


def _build_weights(w_packed, b_packed):
    ws = []
    for li, (cin, cout) in enumerate(_LAYER_DIMS):
        wt = w_packed[3 * li:3 * li + 3, :cout, :cin]
        last = li == len(_LAYER_DIMS) - 1
        om = 3 if last else cout
        m = _align(_F * om, 16)
        kh, seg = _KH[li], _SEG[li]
        w = jnp.zeros((m, _KTOT[li]), jnp.float32)
        for r in range(_F):
            ro = r * om
            for t in range(3):
                rp = r + t - 1
                if 0 <= rp < _F:
                    w = w.at[ro:ro + cout,
                             rp * cin:rp * cin + cin].set(wt[t])
            if r == 0:
                off = kh + ((_F - 1) * cin if li == 0 else 0)
                w = w.at[ro:ro + cout, off:off + cin].set(wt[0])
            if r == _F - 1:
                off = kh + seg + 0
                w = w.at[ro:ro + cout, off:off + cin].set(wt[2])
        bcol = jnp.tile(b_packed[li, :cout, 0], (_F,))
        rows = (om * jnp.arange(_F)[:, None]
                + jnp.arange(cout)[None, :]).reshape(-1)
        w = w.at[rows, kh + 2 * seg].set(bcol)
        ws.append(w.astype(jnp.bfloat16))
    return ws


def _make_body(width):
    bf = jnp.bfloat16
    n_l = len(_LAYER_DIMS)

    def body(x_ref, w1, w2, w3, w4, o_ref, s1, s2, s3, s4):
        wrefs = (w1, w2, w3, w4)
        srefs = (s1, s2, s3, s4)

        @pl.when(pl.program_id(0) == 0)
        def _init():
            for li in range(n_l):
                s = srefs[li]
                kh, seg = _KH[li], _SEG[li]
                s[...] = jnp.zeros(s.shape, bf)
                s[kh + 2 * seg:kh + 2 * seg + 1] = jnp.ones((1, width), bf)

        def put(li, hval):
            s = srefs[li]
            kh, seg = _KH[li], _SEG[li]
            cin = _LAYER_DIMS[li][0]
            s[0:hval.shape[0]] = hval
            if li == 0:
                ep_src, en_src = hval, hval
                ep_rows, en_rows = hval.shape[0], hval.shape[0]
            else:
                ep_src = hval[(_F - 1) * cin:_F * cin]
                en_src = hval[:cin]
                ep_rows, en_rows = cin, cin
            s[kh:kh + ep_rows, _B:] = ep_src[:, :width - _B]
            s[kh + seg:kh + seg + en_rows, :width - _B] = en_src[:, _B:]

        put(0, x_ref[0])
        for li in range(n_l):
            y = jnp.dot(wrefs[li][...], srefs[li][...],
                        preferred_element_type=jnp.float32)
            if _RELUS[li]:
                put(li + 1, jnp.maximum(y.astype(bf), jnp.bfloat16(0)))
            else:
                o_ref[0] = y[:_F * 3].astype(bf)
    return body


_CHUNKS = 1


def _run_chunk(xp, ws, body, c, q, width):
    n_steps = xp.shape[0] // _B
    xt = (xp.astype(jnp.bfloat16)
          .reshape(n_steps, _B, c, q, _F)
          .transpose(0, 4, 2, 3, 1)
          .reshape(n_steps, _F * c, width))

    full = lambda a: pl.BlockSpec(a.shape, lambda i: (0,) * a.ndim)
    out = pl.pallas_call(
        body,
        out_shape=jax.ShapeDtypeStruct((n_steps, _F * c, width), jnp.bfloat16),
        grid=(n_steps,),
        in_specs=[pl.BlockSpec((1, _F * c, width), lambda i: (i, 0, 0))]
        + [full(w) for w in ws],
        out_specs=pl.BlockSpec((1, _F * c, width), lambda i: (i, 0, 0)),
        scratch_shapes=[pltpu.VMEM((k, width), jnp.bfloat16) for k in _KTOT],
        compiler_params=pltpu.CompilerParams(
            dimension_semantics=("arbitrary",),
            vmem_limit_bytes=60 * 1024 * 1024,
        ),
    )(xt, *ws)

    return (out.reshape(n_steps, _F, c, q, _B)
            .transpose(0, 4, 2, 3, 1)
            .reshape(n_steps * _B, c, q * _F))


def kernel(x_ncl, w_packed, b_packed):
    n, c, l = x_ncl.shape
    l_pad = _align(l, _F)
    n_chunk = _B * _CHUNKS
    n_pad = _align(n, n_chunk)

    xp = x_ncl
    if n_pad != n or l_pad != l:
        xp = jnp.pad(x_ncl, ((0, n_pad - n), (0, 0), (0, l_pad - l)))

    q = l_pad // _F
    width = q * _B

    ws = _build_weights(w_packed, b_packed)
    body = _make_body(width)

    csize = n_pad // _CHUNKS
    outs = [_run_chunk(xp[i * csize:(i + 1) * csize], ws, body, c, q, width)
            for i in range(_CHUNKS)]
    out = outs[0] if _CHUNKS == 1 else jnp.concatenate(outs, axis=0)
    out = out.astype(x_ncl.dtype)
    if n_pad != n or l_pad != l:
        out = out[:n, :, :l]
    return out
